# Initial kernel scaffold; baseline (speedup 1.0000x reference)
import jax
import jax.numpy as jnp
from jax.experimental import pallas as pl


def kernel(x, senders, receivers):
    raise NotImplementedError("write your pallas kernel here")


# speedup vs baseline: 2.6815x; 2.6815x over previous
import functools

import jax
import jax.numpy as jnp
from jax import lax
from jax.experimental import pallas as pl
from jax.experimental.pallas import tpu as pltpu
from jax.experimental.pallas import tpu_sc as plsc

N_NODES = 10000
N_EDGES = 320000
D_FEAT = 128

NC = 2
NS = 16
L = 16
NW = NC * NS
EPW = N_EDGES // NW
C = 80
NCHUNK = EPW // C

_mesh = plsc.VectorSubcoreMesh(
    core_axis_name="c", subcore_axis_name="s", num_cores=NC, num_subcores=NS
)


@functools.partial(
    pl.kernel,
    out_type=jax.ShapeDtypeStruct((N_EDGES,), jnp.float32),
    mesh=_mesh,
    scratch_types=[
        pltpu.VMEM((C,), jnp.int32),
        pltpu.VMEM((C,), jnp.int32),
        pltpu.VMEM((C, D_FEAT), jnp.float32),
        pltpu.VMEM((C, D_FEAT), jnp.float32),
        pltpu.VMEM((C,), jnp.float32),
        pltpu.SemaphoreType.DMA,
        pltpu.SemaphoreType.DMA,
    ],
    compiler_params=pltpu.CompilerParams(needs_layout_passes=False),
)
def _node_dot_sc(x_hbm, s_hbm, r_hbm, out_hbm,
                 sidx_v, ridx_v, srows_v, rrows_v, outc_v, sem_s, sem_r):
    wid = lax.axis_index("s") * NC + lax.axis_index("c")
    wbase = wid * EPW

    def chunk_body(c, carry):
        base = wbase + c * C
        pltpu.sync_copy(s_hbm.at[pl.ds(base, C)], sidx_v)
        pltpu.sync_copy(r_hbm.at[pl.ds(base, C)], ridx_v)
        cp_s = pltpu.async_copy(x_hbm.at[sidx_v], srows_v, sem_s)
        cp_r = pltpu.async_copy(x_hbm.at[ridx_v], rrows_v, sem_r)
        cp_s.wait()
        cp_r.wait()
        lanes = lax.iota(jnp.int32, L)

        def group_body(g, carry2):
            res = jnp.zeros((L,), jnp.float32)
            for i in range(L):
                e = g * L + i
                acc = jnp.zeros((L,), jnp.float32)
                for j in range(D_FEAT // L):
                    sv = srows_v[e, pl.ds(j * L, L)]
                    rv = rrows_v[e, pl.ds(j * L, L)]
                    acc = acc + sv * rv
                tot = jnp.sum(acc)
                res = jnp.where(lanes == i, tot, res)
            outc_v[pl.ds(g * L, L)] = res
            return carry2

        lax.fori_loop(0, C // L, group_body, 0)
        pltpu.sync_copy(outc_v, out_hbm.at[pl.ds(base, C)])
        return carry

    lax.fori_loop(0, NCHUNK, chunk_body, 0)


def kernel(x, senders, receivers):
    return _node_dot_sc(x, senders, receivers)


# speedup vs baseline: 7.0773x; 2.6393x over previous
import functools

import jax
import jax.numpy as jnp
from jax import lax
from jax.experimental import pallas as pl
from jax.experimental.pallas import tpu as pltpu
from jax.experimental.pallas import tpu_sc as plsc

N_NODES = 10000
N_EDGES = 320000
D_FEAT = 128

NC = 2
NS = 16
L = 16
NW = NC * NS
EPW = N_EDGES // NW
C = 80
NCHUNK = EPW // C
NGRP = C // L

_mesh = plsc.VectorSubcoreMesh(
    core_axis_name="c", subcore_axis_name="s", num_cores=NC, num_subcores=NS
)


@functools.partial(
    pl.kernel,
    out_type=jax.ShapeDtypeStruct((N_EDGES,), jnp.float32),
    mesh=_mesh,
    scratch_types=[
        pltpu.VMEM((EPW,), jnp.int32),
        pltpu.VMEM((EPW,), jnp.int32),
        pltpu.VMEM((2, C, D_FEAT), jnp.float32),
        pltpu.VMEM((2, C, D_FEAT), jnp.float32),
        pltpu.VMEM((L * L,), jnp.float32),
        pltpu.VMEM((EPW,), jnp.float32),
        pltpu.SemaphoreType.DMA,
        pltpu.SemaphoreType.DMA,
        pltpu.SemaphoreType.DMA,
        pltpu.SemaphoreType.DMA,
    ],
    compiler_params=pltpu.CompilerParams(needs_layout_passes=False),
)
def _node_dot_sc(x_hbm, s_hbm, r_hbm, out_hbm,
                 sidx_v, ridx_v, srows_v, rrows_v, stage_v, out_v,
                 sem_s0, sem_s1, sem_r0, sem_r1):
    wid = lax.axis_index("s") * NC + lax.axis_index("c")
    wbase = wid * EPW
    sems_s = (sem_s0, sem_s1)
    sems_r = (sem_r0, sem_r1)

    pltpu.sync_copy(s_hbm.at[pl.ds(wbase, EPW)], sidx_v)
    pltpu.sync_copy(r_hbm.at[pl.ds(wbase, EPW)], ridx_v)

    def start(c, b):
        pltpu.async_copy(
            x_hbm.at[sidx_v.at[pl.ds(c * C, C)]], srows_v.at[b], sems_s[b])
        pltpu.async_copy(
            x_hbm.at[ridx_v.at[pl.ds(c * C, C)]], rrows_v.at[b], sems_r[b])

    def wait(b):
        pltpu.make_async_copy(
            x_hbm.at[sidx_v.at[pl.ds(0, C)]], srows_v.at[b], sems_s[b]).wait()
        pltpu.make_async_copy(
            x_hbm.at[ridx_v.at[pl.ds(0, C)]], rrows_v.at[b], sems_r[b]).wait()

    lanes16 = lax.iota(jnp.int32, L) * L

    def compute(c, b):
        def group_body(g, carry):
            row0 = g * L
            for i in range(L):
                acc = jnp.zeros((L,), jnp.float32)
                for j in range(D_FEAT // L):
                    sv = srows_v[b, row0 + i, pl.ds(j * L, L)]
                    rv = rrows_v[b, row0 + i, pl.ds(j * L, L)]
                    acc = acc + sv * rv
                stage_v[pl.ds(i * L, L)] = acc
            tot = jnp.zeros((L,), jnp.float32)
            for k in range(L):
                tot = tot + plsc.load_gather(stage_v, [lanes16 + k])
            out_v[pl.ds(c * C + row0, L)] = tot
            return carry

        lax.fori_loop(0, NGRP, group_body, 0)

    start(0, 0)

    def pair_body(p, carry):
        for b in range(2):
            c = 2 * p + b
            start(c + 1, 1 - b)
            wait(b)
            compute(c, b)
        return carry

    lax.fori_loop(0, (NCHUNK - 1) // 2, pair_body, 0)
    wait(0)
    compute(NCHUNK - 1, 0)

    pltpu.sync_copy(out_v, out_hbm.at[pl.ds(wbase, EPW)])


def kernel(x, senders, receivers):
    return _node_dot_sc(x, senders, receivers)
